# baseline (speedup 1.0000x reference)
import functools

import jax
import jax.numpy as jnp
from jax import lax
from jax.experimental import pallas as pl
from jax.experimental.pallas import tpu as pltpu
from jax.experimental.pallas import tpu_sc as plsc

B, S, N, SIZE = 32, 12, 2048, 64
BS = B * S
PAD = 128
ROWS_PER_WORKER = 16
NUM_WORKERS = BS // ROWS_PER_WORKER

_EPS = 1e-5


def _project_body(node_ref, wn_ref, time_ref, wt_ref, day_ref, wd_ref, b_ref,
                  ne_ref, tp_ref, dp_ref):
    ne_ref[...] = jnp.dot(node_ref[...], wn_ref[...],
                          preferred_element_type=jnp.float32)
    tp_ref[...] = jnp.dot(time_ref[...], wt_ref[...],
                          preferred_element_type=jnp.float32)
    dp_ref[...] = jnp.dot(day_ref[...], wd_ref[...],
                          preferred_element_type=jnp.float32) + b_ref[...]


def _project_tables(node_table, W_node, time_table, W_time128, day8, W_day128,
                    b_row128):
    n_nodes = node_table.shape[0]
    n_times = time_table.shape[0]
    return pl.pallas_call(
        _project_body,
        out_shape=(
            jax.ShapeDtypeStruct((n_nodes, SIZE), jnp.float32),
            jax.ShapeDtypeStruct((n_times, PAD), jnp.float32),
            jax.ShapeDtypeStruct((8, PAD), jnp.float32),
        ),
    )(node_table, W_node, time_table, W_time128, day8, W_day128, b_row128)


@functools.lru_cache(maxsize=None)
def _build_gather_combine():
    @functools.partial(
        pl.kernel,
        out_type=jax.ShapeDtypeStruct((BS, PAD), jnp.float32),
        mesh=plsc.VectorSubcoreMesh(core_axis_name="c", subcore_axis_name="s"),
        scratch_types=[
            pltpu.VMEM((ROWS_PER_WORKER,), jnp.int32),
            pltpu.VMEM((ROWS_PER_WORKER,), jnp.int32),
            pltpu.VMEM((ROWS_PER_WORKER, PAD), jnp.float32),
            pltpu.VMEM((ROWS_PER_WORKER, PAD), jnp.float32),
            pltpu.SemaphoreType.DMA,
        ],
    )
    def _gather_combine(tidx_hbm, didx_hbm, tproj_hbm, dproj_hbm, out_hbm,
                        ti_v, di_v, tr_v, dr_v, sem):
        num_cores = 2
        wid = lax.axis_index("s") * num_cores + lax.axis_index("c")

        @pl.when(wid < NUM_WORKERS)
        def _():
            base = wid * ROWS_PER_WORKER
            pltpu.sync_copy(tidx_hbm.at[pl.ds(base, ROWS_PER_WORKER)], ti_v)
            pltpu.sync_copy(didx_hbm.at[pl.ds(base, ROWS_PER_WORKER)], di_v)
            pltpu.async_copy(tproj_hbm.at[ti_v], tr_v, sem).wait()
            pltpu.async_copy(dproj_hbm.at[di_v], dr_v, sem).wait()
            for r in range(ROWS_PER_WORKER):
                for c in range(SIZE // 16):
                    sl = pl.ds(c * 16, 16)
                    tr_v[r, sl] = tr_v[r, sl] + dr_v[r, sl]
            pltpu.sync_copy(tr_v, out_hbm.at[pl.ds(base, ROWS_PER_WORKER)])

    return _gather_combine


def _main_body(d_ref, comb_ref, ne_ref, wrow_ref, g_ref, b_ref, o_ref):
    x = d_ref[...]
    comb = comb_ref[...][:, :SIZE]
    y = x[:, :, None] * wrow_ref[...][None]
    y = y + ne_ref[...][None] + comb[:, None, :]
    mean = jnp.mean(y, axis=-1, keepdims=True)
    c = y - mean
    var = jnp.mean(c * c, axis=-1, keepdims=True)
    o_ref[...] = c * lax.rsqrt(var + _EPS) * g_ref[...][None] + b_ref[...][None]


def _main_pass(data2, combined, node_emb, w_row, g_row, b_row, block_bs):
    grid = (BS // block_bs,)
    return pl.pallas_call(
        _main_body,
        grid=grid,
        in_specs=[
            pl.BlockSpec((block_bs, N), lambda i: (i, 0)),
            pl.BlockSpec((block_bs, PAD), lambda i: (i, 0)),
            pl.BlockSpec((N, SIZE), lambda i: (0, 0)),
            pl.BlockSpec((1, SIZE), lambda i: (0, 0)),
            pl.BlockSpec((1, SIZE), lambda i: (0, 0)),
            pl.BlockSpec((1, SIZE), lambda i: (0, 0)),
        ],
        out_specs=pl.BlockSpec((block_bs, N, SIZE), lambda i: (i, 0, 0)),
        out_shape=jax.ShapeDtypeStruct((BS, N, SIZE), jnp.float32),
    )(data2, combined, node_emb, w_row, g_row, b_row)


def kernel(data, time, weekday, W_data, b_data, node_table, W_node,
           time_table, W_time, day_table, W_day, gamma, beta):
    data2 = data.reshape(BS, N)
    tidx = time.reshape(BS).astype(jnp.int32)
    didx = weekday.reshape(BS).astype(jnp.int32)
    day8 = jnp.zeros((8, day_table.shape[1]), jnp.float32).at[:7].set(day_table)
    pad_cols = PAD - SIZE
    W_time128 = jnp.pad(W_time, ((0, 0), (0, pad_cols)))
    W_day128 = jnp.pad(W_day, ((0, 0), (0, pad_cols)))
    b_row128 = jnp.pad(b_data.reshape(1, SIZE), ((0, 0), (0, pad_cols)))

    node_emb, time_proj, day_proj = _project_tables(
        node_table, W_node, time_table, W_time128, day8, W_day128, b_row128)
    combined = _build_gather_combine()(tidx, didx, time_proj, day_proj)
    out = _main_pass(data2, combined, node_emb, W_data.reshape(1, SIZE),
                     gamma.reshape(1, SIZE), beta.reshape(1, SIZE), block_bs=8)
    return out.reshape(B, S, N, SIZE)
